# Initial kernel scaffold; baseline (speedup 1.0000x reference)
import functools

import jax
import jax.numpy as jnp
from jax import lax
from jax.experimental import pallas as pl
from jax.experimental.pallas import tpu as pltpu
from jax.experimental.pallas import tpu_sc as plsc

B, N, S, K, C = 16, 2048, 512, 24, 128
ST = 64
NST = S // ST
NC, NS = 2, 16
NW = NC * NS
ROWS = B * S * K + B * S
RPW = ROWS // NW
CH = 128
NCH = RPW // CH


def _fps_body(x_ref, y_ref, z_ref, idx_ref, cx_ref, cy_ref, cz_ref):
    X = x_ref[...]
    Y = y_ref[...]
    Z = z_ref[...]
    lane = lax.broadcasted_iota(jnp.int32, (B, N), 1)
    col = lax.broadcasted_iota(jnp.int32, (B, S), 1)

    def step(i, state):
        dists, far, acc_i, acc_x, acc_y, acc_z = state
        m = lane == far
        cx = jnp.sum(jnp.where(m, X, 0.0), axis=1, keepdims=True)
        cy = jnp.sum(jnp.where(m, Y, 0.0), axis=1, keepdims=True)
        cz = jnp.sum(jnp.where(m, Z, 0.0), axis=1, keepdims=True)
        sel = col == i
        acc_i = jnp.where(sel, far, acc_i)
        acc_x = jnp.where(sel, cx, acc_x)
        acc_y = jnp.where(sel, cy, acc_y)
        acc_z = jnp.where(sel, cz, acc_z)
        dx = X - cx
        dy = Y - cy
        dz = Z - cz
        d = dx * dx + dy * dy + dz * dz
        dists = jnp.minimum(dists, d)
        mx = jnp.max(dists, axis=1, keepdims=True)
        far = jnp.min(jnp.where(dists == mx, lane, N), axis=1, keepdims=True)
        return dists, far, acc_i, acc_x, acc_y, acc_z

    dists0 = jnp.full((B, N), 1e10, dtype=jnp.float32)
    far0 = jnp.zeros((B, 1), dtype=jnp.int32)
    acc_i0 = jnp.zeros((B, S), dtype=jnp.int32)
    acc_f0 = jnp.zeros((B, S), dtype=jnp.float32)
    _, _, acc_i, acc_x, acc_y, acc_z = lax.fori_loop(
        0, S, step, (dists0, far0, acc_i0, acc_f0, acc_f0, acc_f0))
    idx_ref[...] = acc_i
    cx_ref[...] = acc_x
    cy_ref[...] = acc_y
    cz_ref[...] = acc_z


def _run_fps(xyz):
    Xc = xyz[:, :, 0]
    Yc = xyz[:, :, 1]
    Zc = xyz[:, :, 2]
    out_types = (
        jax.ShapeDtypeStruct((B, S), jnp.int32),
        jax.ShapeDtypeStruct((B, S), jnp.float32),
        jax.ShapeDtypeStruct((B, S), jnp.float32),
        jax.ShapeDtypeStruct((B, S), jnp.float32),
    )
    return pl.pallas_call(_fps_body, out_shape=out_types)(Xc, Yc, Zc)


def _knn_body(nxyz_ref, xt_ref, idx_ref):
    nx = nxyz_ref[0]
    xt = xt_ref[0]
    src_sq = jnp.sum(nx * nx, axis=1, keepdims=True)
    dst_sq = jnp.sum(xt * xt, axis=0, keepdims=True)
    mm = lax.dot_general(nx, xt, (((1,), (0,)), ((), ())),
                         preferred_element_type=jnp.float32)
    D = -2.0 * mm
    D = D + src_sq
    D = D + dst_sq
    lane = lax.broadcasted_iota(jnp.int32, (S, N), 1)
    colk = lax.broadcasted_iota(jnp.int32, (S, K), 1)

    def step(k, state):
        D, acc = state
        mn = jnp.min(D, axis=1, keepdims=True)
        j = jnp.min(jnp.where(D == mn, lane, N), axis=1, keepdims=True)
        acc = jnp.where(colk == k, j, acc)
        D = jnp.where(lane == j, jnp.float32(jnp.inf), D)
        return D, acc

    acc0 = jnp.zeros((S, K), dtype=jnp.int32)
    _, acc = lax.fori_loop(0, K, step, (D, acc0))
    idx_ref[0] = acc


def _run_knn(new_xyz, xt):
    return pl.pallas_call(
        _knn_body,
        grid=(B,),
        in_specs=[
            pl.BlockSpec((1, S, 3), lambda b: (b, 0, 0)),
            pl.BlockSpec((1, 3, N), lambda b: (b, 0, 0)),
        ],
        out_specs=pl.BlockSpec((1, S, K), lambda b: (b, 0, 0)),
        out_shape=jax.ShapeDtypeStruct((B, S, K), jnp.int32),
    )(new_xyz, xt)


def _sc_gather_rows(tbl, gidx3):
    mesh = plsc.VectorSubcoreMesh(core_axis_name="c", subcore_axis_name="s")

    @functools.partial(
        pl.kernel,
        mesh=mesh,
        out_type=jax.ShapeDtypeStruct((ROWS, C), jnp.float32),
        scratch_types=[
            pltpu.VMEM((NCH, CH), jnp.int32),
            pltpu.VMEM((CH, C), jnp.float32),
            pltpu.SemaphoreType.DMA,
        ],
    )
    def gather_kernel(tbl_hbm, gidx_hbm, out_hbm, idx_v, buf, sem):
        wid = lax.axis_index("s") * NC + lax.axis_index("c")
        base = wid * RPW
        pltpu.sync_copy(gidx_hbm.at[wid], idx_v)

        def chunk(c, carry):
            pltpu.async_copy(tbl_hbm.at[idx_v.at[c]], buf, sem).wait()
            pltpu.sync_copy(buf, out_hbm.at[pl.ds(base + c * CH, CH)])
            return carry

        lax.fori_loop(0, NCH, chunk, 0)

    return gather_kernel(tbl, gidx3)


def _stats_body(g_ref, ss_ref, sm_ref):
    s_idx = pl.program_id(1)

    @pl.when(s_idx == 0)
    def _():
        ss_ref[...] = jnp.zeros((1, 1), jnp.float32)
        sm_ref[...] = jnp.zeros((1, 1), jnp.float32)

    g = g_ref[0]
    mean = jnp.mean(g, axis=1)
    ss_ref[...] += jnp.sum(g * g).reshape(1, 1)
    sm_ref[...] += jnp.sum(mean * mean).reshape(1, 1)


def _run_stats(grouped):
    return pl.pallas_call(
        _stats_body,
        grid=(B, NST),
        in_specs=[pl.BlockSpec((1, ST, K, C), lambda b, s: (b, s, 0, 0))],
        out_specs=(
            pl.BlockSpec((1, 1), lambda b, s: (b, 0)),
            pl.BlockSpec((1, 1), lambda b, s: (b, 0)),
        ),
        out_shape=(
            jax.ShapeDtypeStruct((B, 1), jnp.float32),
            jax.ShapeDtypeStruct((B, 1), jnp.float32),
        ),
    )(grouped)


def _norm_body(g_ref, np_ref, sp_ref, al_ref, be_ref, out_ref):
    g = g_ref[0]
    mean = jnp.mean(g, axis=1, keepdims=True)
    stdpe = sp_ref[0, 0]
    gp = (g - mean) / stdpe
    al = al_ref[...].reshape(1, 1, C)
    be = be_ref[...].reshape(1, 1, C)
    gp = al * gp + be
    out_ref[0, :, :, 0:C] = gp
    rep = np_ref[0]
    out_ref[0, :, :, C:2 * C] = jnp.broadcast_to(rep[:, None, :], (ST, K, C))


def _run_norm(grouped, new_p, stdpe, alpha, beta):
    return pl.pallas_call(
        _norm_body,
        grid=(B, NST),
        in_specs=[
            pl.BlockSpec((1, ST, K, C), lambda b, s: (b, s, 0, 0)),
            pl.BlockSpec((1, ST, C), lambda b, s: (b, s, 0)),
            pl.BlockSpec((1, 1), lambda b, s: (b, 0),
                         memory_space=pltpu.SMEM),
            pl.BlockSpec((1, C), lambda b, s: (0, 0)),
            pl.BlockSpec((1, C), lambda b, s: (0, 0)),
        ],
        out_specs=pl.BlockSpec((1, ST, K, 2 * C), lambda b, s: (b, s, 0, 0)),
        out_shape=jax.ShapeDtypeStruct((B, S, K, 2 * C), jnp.float32),
    )(grouped, new_p, stdpe, alpha, beta)


def kernel(xyz, p, affine_alpha, affine_beta):
    fps_idx, cx, cy, cz = _run_fps(xyz)
    new_xyz = jnp.stack([cx, cy, cz], axis=-1)

    xt = jnp.transpose(xyz, (0, 2, 1))
    idx = _run_knn(new_xyz, xt)

    boff = (jnp.arange(B, dtype=jnp.int32) * N)
    knn_rows = (idx + boff[:, None, None]).reshape(-1)
    fps_rows = (fps_idx + boff[:, None]).reshape(-1)
    gidx = jnp.concatenate([knn_rows, fps_rows])
    gidx3 = gidx.reshape(NW, NCH, CH)

    tbl = p.reshape(B * N, C)
    rows = _sc_gather_rows(tbl, gidx3)
    grouped = rows[: B * S * K].reshape(B, S, K, C)
    new_p = rows[B * S * K:].reshape(B, S, C)

    ss, sm = _run_stats(grouped)
    var = (ss - jnp.float32(K) * sm) / jnp.float32(S * K * C - 1)
    stdpe = jnp.sqrt(var) + jnp.float32(1e-5)

    al = affine_alpha.reshape(1, C)
    be = affine_beta.reshape(1, C)
    new_p_out = _run_norm(grouped, new_p, stdpe, al, be)
    return (new_xyz, new_p_out)


# speedup vs baseline: 9.0363x; 9.0363x over previous
import functools

import jax
import jax.numpy as jnp
from jax import lax
from jax.experimental import pallas as pl
from jax.experimental.pallas import tpu as pltpu
from jax.experimental.pallas import tpu_sc as plsc

B, N, S, K, C = 16, 2048, 512, 24, 128
ST = 64
NST = S // ST
NC, NS = 2, 16
NW = NC * NS
ROWS = B * S * K + B * S
RPW = ROWS // NW
CH = 128
NCH = RPW // CH


def _fps_body(x_ref, y_ref, z_ref, idx_ref, cx_ref, cy_ref, cz_ref):
    X = x_ref[...]
    Y = y_ref[...]
    Z = z_ref[...]
    lane = lax.broadcasted_iota(jnp.int32, (B, N), 1)
    col = lax.broadcasted_iota(jnp.int32, (B, S), 1)

    idx_ref[...] = jnp.zeros((B, S), jnp.int32)
    cx_ref[...] = jnp.zeros((B, S), jnp.float32)
    cy_ref[...] = jnp.zeros((B, S), jnp.float32)
    cz_ref[...] = jnp.zeros((B, S), jnp.float32)

    def step(i, state):
        dists, far = state
        m = lane == far
        cx = jnp.sum(jnp.where(m, X, 0.0), axis=1, keepdims=True)
        cy = jnp.sum(jnp.where(m, Y, 0.0), axis=1, keepdims=True)
        cz = jnp.sum(jnp.where(m, Z, 0.0), axis=1, keepdims=True)
        sel = col == i
        idx_ref[...] = jnp.where(sel, far, idx_ref[...])
        cx_ref[...] = jnp.where(sel, cx, cx_ref[...])
        cy_ref[...] = jnp.where(sel, cy, cy_ref[...])
        cz_ref[...] = jnp.where(sel, cz, cz_ref[...])
        dx = X - cx
        dy = Y - cy
        dz = Z - cz
        d = dx * dx + dy * dy + dz * dz
        dists = jnp.minimum(dists, d)
        mx = jnp.max(dists, axis=1, keepdims=True)
        far = jnp.min(jnp.where(dists == mx, lane, N), axis=1, keepdims=True)
        return dists, far

    dists0 = X * 0.0 + jnp.float32(1e10)
    far0 = jnp.zeros((B, 1), dtype=jnp.int32)
    lax.fori_loop(0, S, step, (dists0, far0))


def _run_fps(xyz):
    Xc = xyz[:, :, 0]
    Yc = xyz[:, :, 1]
    Zc = xyz[:, :, 2]
    out_types = (
        jax.ShapeDtypeStruct((B, S), jnp.int32),
        jax.ShapeDtypeStruct((B, S), jnp.float32),
        jax.ShapeDtypeStruct((B, S), jnp.float32),
        jax.ShapeDtypeStruct((B, S), jnp.float32),
    )
    return pl.pallas_call(_fps_body, out_shape=out_types)(Xc, Yc, Zc)


def _knn_body(nxyz_ref, xt_ref, idx_ref):
    nx = nxyz_ref[0]
    xt = xt_ref[0]
    src_sq = jnp.sum(nx * nx, axis=1, keepdims=True)
    dst_sq = jnp.sum(xt * xt, axis=0, keepdims=True)
    mm = lax.dot_general(nx, xt, (((1,), (0,)), ((), ())),
                         preferred_element_type=jnp.float32)
    D = -2.0 * mm
    D = D + src_sq
    D = D + dst_sq
    lane = lax.broadcasted_iota(jnp.int32, (S, N), 1)
    colk = lax.broadcasted_iota(jnp.int32, (S, K), 1)

    idx_ref[0] = jnp.zeros((S, K), dtype=jnp.int32)

    def step(k, D):
        mn = jnp.min(D, axis=1, keepdims=True)
        j = jnp.min(jnp.where(D == mn, lane, N), axis=1, keepdims=True)
        idx_ref[0] = jnp.where(colk == k, j, idx_ref[0])
        D = jnp.where(lane == j, jnp.float32(jnp.inf), D)
        return D

    lax.fori_loop(0, K, step, D)


def _run_knn(new_xyz, xt):
    return pl.pallas_call(
        _knn_body,
        grid=(B,),
        in_specs=[
            pl.BlockSpec((1, S, 3), lambda b: (b, 0, 0)),
            pl.BlockSpec((1, 3, N), lambda b: (b, 0, 0)),
        ],
        out_specs=pl.BlockSpec((1, S, K), lambda b: (b, 0, 0)),
        out_shape=jax.ShapeDtypeStruct((B, S, K), jnp.int32),
    )(new_xyz, xt)


def _sc_gather_rows(tbl, gidx3):
    mesh = plsc.VectorSubcoreMesh(core_axis_name="c", subcore_axis_name="s")

    @functools.partial(
        pl.kernel,
        mesh=mesh,
        out_type=jax.ShapeDtypeStruct((ROWS, C), jnp.float32),
        scratch_types=[
            pltpu.VMEM((NCH, CH), jnp.int32),
            pltpu.VMEM((CH, C), jnp.float32),
            pltpu.SemaphoreType.DMA,
        ],
    )
    def gather_kernel(tbl_hbm, gidx_hbm, out_hbm, idx_v, buf, sem):
        wid = lax.axis_index("s") * NC + lax.axis_index("c")
        base = wid * RPW
        pltpu.sync_copy(gidx_hbm.at[wid], idx_v)

        def chunk(c, carry):
            pltpu.async_copy(tbl_hbm.at[idx_v.at[c]], buf, sem).wait()
            pltpu.sync_copy(buf, out_hbm.at[pl.ds(base + c * CH, CH)])
            return carry

        lax.fori_loop(0, NCH, chunk, 0)

    return gather_kernel(tbl, gidx3)


def _stats_body(g_ref, ss_ref, sm_ref):
    s_idx = pl.program_id(1)

    @pl.when(s_idx == 0)
    def _():
        ss_ref[...] = jnp.zeros((1, 1, 1), jnp.float32)
        sm_ref[...] = jnp.zeros((1, 1, 1), jnp.float32)

    g = g_ref[0]
    mean = jnp.mean(g, axis=1)
    ss_ref[...] += jnp.sum(g * g).reshape(1, 1, 1)
    sm_ref[...] += jnp.sum(mean * mean).reshape(1, 1, 1)


def _run_stats(grouped):
    return pl.pallas_call(
        _stats_body,
        grid=(B, NST),
        in_specs=[pl.BlockSpec((1, ST, K, C), lambda b, s: (b, s, 0, 0))],
        out_specs=(
            pl.BlockSpec((1, 1, 1), lambda b, s: (b, 0, 0)),
            pl.BlockSpec((1, 1, 1), lambda b, s: (b, 0, 0)),
        ),
        out_shape=(
            jax.ShapeDtypeStruct((B, 1, 1), jnp.float32),
            jax.ShapeDtypeStruct((B, 1, 1), jnp.float32),
        ),
    )(grouped)


def _norm_body(g_ref, np_ref, sp_ref, al_ref, be_ref, out_ref):
    g = g_ref[0]
    mean = jnp.mean(g, axis=1, keepdims=True)
    stdpe = sp_ref[0, 0, 0]
    gp = (g - mean) / stdpe
    al = al_ref[...].reshape(1, 1, C)
    be = be_ref[...].reshape(1, 1, C)
    gp = al * gp + be
    out_ref[0, :, :, 0:C] = gp
    rep = np_ref[0]
    out_ref[0, :, :, C:2 * C] = jnp.broadcast_to(rep[:, None, :], (ST, K, C))


def _run_norm(grouped, new_p, stdpe, alpha, beta):
    return pl.pallas_call(
        _norm_body,
        grid=(B, NST),
        in_specs=[
            pl.BlockSpec((1, ST, K, C), lambda b, s: (b, s, 0, 0)),
            pl.BlockSpec((1, ST, C), lambda b, s: (b, s, 0)),
            pl.BlockSpec((1, 1, 1), lambda b, s: (b, 0, 0),
                         memory_space=pltpu.SMEM),
            pl.BlockSpec((1, C), lambda b, s: (0, 0)),
            pl.BlockSpec((1, C), lambda b, s: (0, 0)),
        ],
        out_specs=pl.BlockSpec((1, ST, K, 2 * C), lambda b, s: (b, s, 0, 0)),
        out_shape=jax.ShapeDtypeStruct((B, S, K, 2 * C), jnp.float32),
    )(grouped, new_p, stdpe, alpha, beta)


def kernel(xyz, p, affine_alpha, affine_beta):
    fps_idx, cx, cy, cz = _run_fps(xyz)
    new_xyz = jnp.stack([cx, cy, cz], axis=-1)

    xt = jnp.transpose(xyz, (0, 2, 1))
    idx = _run_knn(new_xyz, xt)

    boff = (jnp.arange(B, dtype=jnp.int32) * N)
    knn_rows = (idx + boff[:, None, None]).reshape(-1)
    fps_rows = (fps_idx + boff[:, None]).reshape(-1)
    gidx = jnp.concatenate([knn_rows, fps_rows])
    gidx3 = gidx.reshape(NW, NCH, CH)

    tbl = p.reshape(B * N, C)
    rows = _sc_gather_rows(tbl, gidx3)
    grouped = rows[: B * S * K].reshape(B, S, K, C)
    new_p = rows[B * S * K:].reshape(B, S, C)

    ss, sm = _run_stats(grouped)
    var = (ss - jnp.float32(K) * sm) / jnp.float32(S * K * C - 1)
    stdpe = jnp.sqrt(var) + jnp.float32(1e-5)

    al = affine_alpha.reshape(1, C)
    be = affine_beta.reshape(1, C)
    new_p_out = _run_norm(grouped, new_p, stdpe, al, be)
    return (new_xyz, new_p_out)


# speedup vs baseline: 10.8626x; 1.2021x over previous
import functools

import jax
import jax.numpy as jnp
from jax import lax
from jax.experimental import pallas as pl
from jax.experimental.pallas import tpu as pltpu
from jax.experimental.pallas import tpu_sc as plsc

B, N, S, K, C = 16, 2048, 512, 24, 128
ST = 64
NST = S // ST
NC, NS = 2, 16
NW = NC * NS
ROWS = B * S * K + B * S
RPW = ROWS // NW
CH = 128
NCH = RPW // CH


def _fps_body(x_ref, y_ref, z_ref, idx_ref, cx_ref, cy_ref, cz_ref):
    X = x_ref[...]
    Y = y_ref[...]
    Z = z_ref[...]
    lane = lax.broadcasted_iota(jnp.int32, (B, N), 1)
    col = lax.broadcasted_iota(jnp.int32, (B, S), 1)

    idx_ref[...] = jnp.zeros((B, S), jnp.int32)
    cx_ref[...] = jnp.zeros((B, S), jnp.float32)
    cy_ref[...] = jnp.zeros((B, S), jnp.float32)
    cz_ref[...] = jnp.zeros((B, S), jnp.float32)

    def step(i, state):
        dists, far = state
        m = lane == far
        cx = jnp.sum(jnp.where(m, X, 0.0), axis=1, keepdims=True)
        cy = jnp.sum(jnp.where(m, Y, 0.0), axis=1, keepdims=True)
        cz = jnp.sum(jnp.where(m, Z, 0.0), axis=1, keepdims=True)
        sel = col == i
        idx_ref[...] = jnp.where(sel, far, idx_ref[...])
        cx_ref[...] = jnp.where(sel, cx, cx_ref[...])
        cy_ref[...] = jnp.where(sel, cy, cy_ref[...])
        cz_ref[...] = jnp.where(sel, cz, cz_ref[...])
        dx = X - cx
        dy = Y - cy
        dz = Z - cz
        d = dx * dx + dy * dy + dz * dz
        dists = jnp.minimum(dists, d)
        mx = jnp.max(dists, axis=1, keepdims=True)
        far = jnp.min(jnp.where(dists == mx, lane, N), axis=1, keepdims=True)
        return dists, far

    dists0 = X * 0.0 + jnp.float32(1e10)
    far0 = jnp.zeros((B, 1), dtype=jnp.int32)
    lax.fori_loop(0, S, step, (dists0, far0))


def _run_fps(xyz):
    Xc = xyz[:, :, 0]
    Yc = xyz[:, :, 1]
    Zc = xyz[:, :, 2]
    out_types = (
        jax.ShapeDtypeStruct((B, S), jnp.int32),
        jax.ShapeDtypeStruct((B, S), jnp.float32),
        jax.ShapeDtypeStruct((B, S), jnp.float32),
        jax.ShapeDtypeStruct((B, S), jnp.float32),
    )
    return pl.pallas_call(_fps_body, out_shape=out_types)(Xc, Yc, Zc)


def _knn_body(nxyz_ref, xt_ref, idx_ref, d_ref, io_ref):
    nx = nxyz_ref[0]
    xt = xt_ref[0]
    src_sq = jnp.sum(nx * nx, axis=1, keepdims=True)
    dst_sq = jnp.sum(xt * xt, axis=0, keepdims=True)
    mm = lax.dot_general(nx, xt, (((1,), (0,)), ((), ())),
                         preferred_element_type=jnp.float32)
    D = -2.0 * mm
    D = D + src_sq
    D = D + dst_sq
    d_ref[...] = D
    io_ref[...] = lax.broadcasted_iota(jnp.int32, (S, N), 1)
    colk = lax.broadcasted_iota(jnp.int32, (S, K), 1)

    idx_ref[0] = jnp.zeros((S, K), dtype=jnp.int32)

    def step(k, jprev):
        lane = io_ref[...]
        D = d_ref[...]
        D = jnp.where(lane == jprev, jnp.float32(jnp.inf), D)
        d_ref[...] = D
        mn = jnp.min(D, axis=1, keepdims=True)
        j = jnp.min(jnp.where(D == mn, lane, N), axis=1, keepdims=True)
        idx_ref[0] = jnp.where(colk == k, j, idx_ref[0])
        return j

    lax.fori_loop(0, K, step, jnp.full((S, 1), -1, jnp.int32))


def _run_knn(new_xyz, xt):
    return pl.pallas_call(
        _knn_body,
        grid=(B,),
        in_specs=[
            pl.BlockSpec((1, S, 3), lambda b: (b, 0, 0)),
            pl.BlockSpec((1, 3, N), lambda b: (b, 0, 0)),
        ],
        out_specs=pl.BlockSpec((1, S, K), lambda b: (b, 0, 0)),
        out_shape=jax.ShapeDtypeStruct((B, S, K), jnp.int32),
        scratch_shapes=[
            pltpu.VMEM((S, N), jnp.float32),
            pltpu.VMEM((S, N), jnp.int32),
        ],
    )(new_xyz, xt)


def _sc_gather_rows(tbl, gidx3):
    mesh = plsc.VectorSubcoreMesh(core_axis_name="c", subcore_axis_name="s")

    @functools.partial(
        pl.kernel,
        mesh=mesh,
        out_type=jax.ShapeDtypeStruct((ROWS, C), jnp.float32),
        scratch_types=[
            pltpu.VMEM((NCH, CH), jnp.int32),
            pltpu.VMEM((CH, C), jnp.float32),
            pltpu.SemaphoreType.DMA,
        ],
    )
    def gather_kernel(tbl_hbm, gidx_hbm, out_hbm, idx_v, buf, sem):
        wid = lax.axis_index("s") * NC + lax.axis_index("c")
        base = wid * RPW
        pltpu.sync_copy(gidx_hbm.at[wid], idx_v)

        def chunk(c, carry):
            pltpu.async_copy(tbl_hbm.at[idx_v.at[c]], buf, sem).wait()
            pltpu.sync_copy(buf, out_hbm.at[pl.ds(base + c * CH, CH)])
            return carry

        lax.fori_loop(0, NCH, chunk, 0)

    return gather_kernel(tbl, gidx3)


def _stats_body(g_ref, ss_ref, sm_ref):
    s_idx = pl.program_id(1)

    @pl.when(s_idx == 0)
    def _():
        ss_ref[...] = jnp.zeros((1, 1, 1), jnp.float32)
        sm_ref[...] = jnp.zeros((1, 1, 1), jnp.float32)

    g = g_ref[0]
    mean = jnp.mean(g, axis=1)
    ss_ref[...] += jnp.sum(g * g).reshape(1, 1, 1)
    sm_ref[...] += jnp.sum(mean * mean).reshape(1, 1, 1)


def _run_stats(grouped):
    return pl.pallas_call(
        _stats_body,
        grid=(B, NST),
        in_specs=[pl.BlockSpec((1, ST, K, C), lambda b, s: (b, s, 0, 0))],
        out_specs=(
            pl.BlockSpec((1, 1, 1), lambda b, s: (b, 0, 0)),
            pl.BlockSpec((1, 1, 1), lambda b, s: (b, 0, 0)),
        ),
        out_shape=(
            jax.ShapeDtypeStruct((B, 1, 1), jnp.float32),
            jax.ShapeDtypeStruct((B, 1, 1), jnp.float32),
        ),
    )(grouped)


def _norm_body(g_ref, np_ref, sp_ref, al_ref, be_ref, out_ref):
    g = g_ref[0]
    mean = jnp.mean(g, axis=1, keepdims=True)
    stdpe = sp_ref[0, 0, 0]
    gp = (g - mean) / stdpe
    al = al_ref[...].reshape(1, 1, C)
    be = be_ref[...].reshape(1, 1, C)
    gp = al * gp + be
    out_ref[0, :, :, 0:C] = gp
    rep = np_ref[0]
    out_ref[0, :, :, C:2 * C] = jnp.broadcast_to(rep[:, None, :], (ST, K, C))


def _run_norm(grouped, new_p, stdpe, alpha, beta):
    return pl.pallas_call(
        _norm_body,
        grid=(B, NST),
        in_specs=[
            pl.BlockSpec((1, ST, K, C), lambda b, s: (b, s, 0, 0)),
            pl.BlockSpec((1, ST, C), lambda b, s: (b, s, 0)),
            pl.BlockSpec((1, 1, 1), lambda b, s: (b, 0, 0),
                         memory_space=pltpu.SMEM),
            pl.BlockSpec((1, C), lambda b, s: (0, 0)),
            pl.BlockSpec((1, C), lambda b, s: (0, 0)),
        ],
        out_specs=pl.BlockSpec((1, ST, K, 2 * C), lambda b, s: (b, s, 0, 0)),
        out_shape=jax.ShapeDtypeStruct((B, S, K, 2 * C), jnp.float32),
    )(grouped, new_p, stdpe, alpha, beta)


def kernel(xyz, p, affine_alpha, affine_beta):
    fps_idx, cx, cy, cz = _run_fps(xyz)
    new_xyz = jnp.stack([cx, cy, cz], axis=-1)

    xt = jnp.transpose(xyz, (0, 2, 1))
    idx = _run_knn(new_xyz, xt)

    boff = (jnp.arange(B, dtype=jnp.int32) * N)
    knn_rows = (idx + boff[:, None, None]).reshape(-1)
    fps_rows = (fps_idx + boff[:, None]).reshape(-1)
    gidx = jnp.concatenate([knn_rows, fps_rows])
    gidx3 = gidx.reshape(NW, NCH, CH)

    tbl = p.reshape(B * N, C)
    rows = _sc_gather_rows(tbl, gidx3)
    grouped = rows[: B * S * K].reshape(B, S, K, C)
    new_p = rows[B * S * K:].reshape(B, S, C)

    ss, sm = _run_stats(grouped)
    var = (ss - jnp.float32(K) * sm) / jnp.float32(S * K * C - 1)
    stdpe = jnp.sqrt(var) + jnp.float32(1e-5)

    al = affine_alpha.reshape(1, C)
    be = affine_beta.reshape(1, C)
    new_p_out = _run_norm(grouped, new_p, stdpe, al, be)
    return (new_xyz, new_p_out)


# speedup vs baseline: 11.5730x; 1.0654x over previous
import functools

import jax
import jax.numpy as jnp
from jax import lax
from jax.experimental import pallas as pl
from jax.experimental.pallas import tpu as pltpu
from jax.experimental.pallas import tpu_sc as plsc

B, N, S, K, C = 16, 2048, 512, 24, 128
ST = 128
NST = S // ST
NC, NS = 2, 16
NW = NC * NS
ROWS = B * S * K + B * S
RPW = ROWS // NW
CH = 128
NCH = RPW // CH


def _fps_body(x_ref, y_ref, z_ref, idx_ref, cx_ref, cy_ref, cz_ref):
    X = x_ref[...]
    Y = y_ref[...]
    Z = z_ref[...]
    lane = lax.broadcasted_iota(jnp.int32, (B, N), 1)
    col = lax.broadcasted_iota(jnp.int32, (B, S), 1)

    idx_ref[...] = jnp.zeros((B, S), jnp.int32)
    cx_ref[...] = jnp.zeros((B, S), jnp.float32)
    cy_ref[...] = jnp.zeros((B, S), jnp.float32)
    cz_ref[...] = jnp.zeros((B, S), jnp.float32)

    def step(i, state):
        dists, far = state
        m = lane == far
        cx = jnp.sum(jnp.where(m, X, 0.0), axis=1, keepdims=True)
        cy = jnp.sum(jnp.where(m, Y, 0.0), axis=1, keepdims=True)
        cz = jnp.sum(jnp.where(m, Z, 0.0), axis=1, keepdims=True)
        sel = col == i
        idx_ref[...] = jnp.where(sel, far, idx_ref[...])
        cx_ref[...] = jnp.where(sel, cx, cx_ref[...])
        cy_ref[...] = jnp.where(sel, cy, cy_ref[...])
        cz_ref[...] = jnp.where(sel, cz, cz_ref[...])
        dx = X - cx
        dy = Y - cy
        dz = Z - cz
        d = dx * dx + dy * dy + dz * dz
        dists = jnp.minimum(dists, d)
        mx = jnp.max(dists, axis=1, keepdims=True)
        far = jnp.min(jnp.where(dists == mx, lane, N), axis=1, keepdims=True)
        return dists, far

    dists0 = X * 0.0 + jnp.float32(1e10)
    far0 = jnp.zeros((B, 1), dtype=jnp.int32)
    lax.fori_loop(0, S, step, (dists0, far0))


def _run_fps(xyz):
    Xc = xyz[:, :, 0]
    Yc = xyz[:, :, 1]
    Zc = xyz[:, :, 2]
    out_types = (
        jax.ShapeDtypeStruct((B, S), jnp.int32),
        jax.ShapeDtypeStruct((B, S), jnp.float32),
        jax.ShapeDtypeStruct((B, S), jnp.float32),
        jax.ShapeDtypeStruct((B, S), jnp.float32),
    )
    return pl.pallas_call(_fps_body, out_shape=out_types)(Xc, Yc, Zc)


def _knn_body(nxyz_ref, xt_ref, idx_ref, d_ref, io_ref):
    nx = nxyz_ref[0]
    xt = xt_ref[0]
    src_sq = jnp.sum(nx * nx, axis=1, keepdims=True)
    dst_sq = jnp.sum(xt * xt, axis=0, keepdims=True)
    mm = lax.dot_general(nx, xt, (((1,), (0,)), ((), ())),
                         preferred_element_type=jnp.float32)
    D = -2.0 * mm
    D = D + src_sq
    D = D + dst_sq
    d_ref[...] = D
    io_ref[...] = lax.broadcasted_iota(jnp.int32, (S, N), 1)
    colk = lax.broadcasted_iota(jnp.int32, (S, K), 1)

    idx_ref[0] = jnp.zeros((S, K), dtype=jnp.int32)

    def step(k, jprev):
        lane = io_ref[...]
        D = d_ref[...]
        D = jnp.where(lane == jprev, jnp.float32(jnp.inf), D)
        d_ref[...] = D
        mn = jnp.min(D, axis=1, keepdims=True)
        j = jnp.min(jnp.where(D == mn, lane, N), axis=1, keepdims=True)
        idx_ref[0] = jnp.where(colk == k, j, idx_ref[0])
        return j

    lax.fori_loop(0, K, step, jnp.full((S, 1), -1, jnp.int32))


def _run_knn(new_xyz, xt):
    return pl.pallas_call(
        _knn_body,
        grid=(B,),
        in_specs=[
            pl.BlockSpec((1, S, 3), lambda b: (b, 0, 0)),
            pl.BlockSpec((1, 3, N), lambda b: (b, 0, 0)),
        ],
        out_specs=pl.BlockSpec((1, S, K), lambda b: (b, 0, 0)),
        out_shape=jax.ShapeDtypeStruct((B, S, K), jnp.int32),
        scratch_shapes=[
            pltpu.VMEM((S, N), jnp.float32),
            pltpu.VMEM((S, N), jnp.int32),
        ],
    )(new_xyz, xt)


def _sc_gather_rows(tbl, gidx3):
    mesh = plsc.VectorSubcoreMesh(core_axis_name="c", subcore_axis_name="s")

    @functools.partial(
        pl.kernel,
        mesh=mesh,
        out_type=jax.ShapeDtypeStruct((ROWS, C), jnp.float32),
        scratch_types=[
            pltpu.VMEM((NCH, CH), jnp.int32),
            pltpu.VMEM((CH, C), jnp.float32),
            pltpu.SemaphoreType.DMA,
        ],
    )
    def gather_kernel(tbl_hbm, gidx_hbm, out_hbm, idx_v, buf, sem):
        wid = lax.axis_index("s") * NC + lax.axis_index("c")
        base = wid * RPW
        pltpu.sync_copy(gidx_hbm.at[wid], idx_v)

        def chunk(c, carry):
            pltpu.async_copy(tbl_hbm.at[idx_v.at[c]], buf, sem).wait()
            pltpu.sync_copy(buf, out_hbm.at[pl.ds(base + c * CH, CH)])
            return carry

        lax.fori_loop(0, NCH, chunk, 0)

    return gather_kernel(tbl, gidx3)


def _stats_body(g_ref, ss_ref, sm_ref):
    s_idx = pl.program_id(1)

    @pl.when(s_idx == 0)
    def _():
        ss_ref[...] = jnp.zeros((1, 1, 1), jnp.float32)
        sm_ref[...] = jnp.zeros((1, 1, 1), jnp.float32)

    g = g_ref[0]
    mean = jnp.mean(g, axis=1)
    ss_ref[...] += jnp.sum(g * g).reshape(1, 1, 1)
    sm_ref[...] += jnp.sum(mean * mean).reshape(1, 1, 1)


def _run_stats(grouped):
    return pl.pallas_call(
        _stats_body,
        grid=(B, NST),
        in_specs=[pl.BlockSpec((1, ST, K, C), lambda b, s: (b, s, 0, 0))],
        out_specs=(
            pl.BlockSpec((1, 1, 1), lambda b, s: (b, 0, 0)),
            pl.BlockSpec((1, 1, 1), lambda b, s: (b, 0, 0)),
        ),
        out_shape=(
            jax.ShapeDtypeStruct((B, 1, 1), jnp.float32),
            jax.ShapeDtypeStruct((B, 1, 1), jnp.float32),
        ),
    )(grouped)


def _norm_body(g_ref, np_ref, sp_ref, al_ref, be_ref, out_ref):
    g = g_ref[0]
    mean = jnp.mean(g, axis=1, keepdims=True)
    stdpe = sp_ref[0, 0, 0]
    gp = (g - mean) / stdpe
    al = al_ref[...].reshape(1, 1, C)
    be = be_ref[...].reshape(1, 1, C)
    gp = al * gp + be
    out_ref[0, :, :, 0:C] = gp
    rep = np_ref[0]
    out_ref[0, :, :, C:2 * C] = jnp.broadcast_to(rep[:, None, :], (ST, K, C))


def _run_norm(grouped, new_p, stdpe, alpha, beta):
    return pl.pallas_call(
        _norm_body,
        grid=(B, NST),
        in_specs=[
            pl.BlockSpec((1, ST, K, C), lambda b, s: (b, s, 0, 0)),
            pl.BlockSpec((1, ST, C), lambda b, s: (b, s, 0)),
            pl.BlockSpec((1, 1, 1), lambda b, s: (b, 0, 0),
                         memory_space=pltpu.SMEM),
            pl.BlockSpec((1, C), lambda b, s: (0, 0)),
            pl.BlockSpec((1, C), lambda b, s: (0, 0)),
        ],
        out_specs=pl.BlockSpec((1, ST, K, 2 * C), lambda b, s: (b, s, 0, 0)),
        out_shape=jax.ShapeDtypeStruct((B, S, K, 2 * C), jnp.float32),
    )(grouped, new_p, stdpe, alpha, beta)


def kernel(xyz, p, affine_alpha, affine_beta):
    fps_idx, cx, cy, cz = _run_fps(xyz)
    new_xyz = jnp.stack([cx, cy, cz], axis=-1)

    xt = jnp.transpose(xyz, (0, 2, 1))
    idx = _run_knn(new_xyz, xt)

    boff = (jnp.arange(B, dtype=jnp.int32) * N)
    knn_rows = (idx + boff[:, None, None]).reshape(-1)
    fps_rows = (fps_idx + boff[:, None]).reshape(-1)
    gidx = jnp.concatenate([knn_rows, fps_rows])
    gidx3 = gidx.reshape(NW, NCH, CH)

    tbl = p.reshape(B * N, C)
    rows = _sc_gather_rows(tbl, gidx3)
    grouped = rows[: B * S * K].reshape(B, S, K, C)
    new_p = rows[B * S * K:].reshape(B, S, C)

    ss, sm = _run_stats(grouped)
    var = (ss - jnp.float32(K) * sm) / jnp.float32(S * K * C - 1)
    stdpe = jnp.sqrt(var) + jnp.float32(1e-5)

    al = affine_alpha.reshape(1, C)
    be = affine_beta.reshape(1, C)
    new_p_out = _run_norm(grouped, new_p, stdpe, al, be)
    return (new_xyz, new_p_out)


# speedup vs baseline: 11.9438x; 1.0320x over previous
import functools

import jax
import jax.numpy as jnp
from jax import lax
from jax.experimental import pallas as pl
from jax.experimental.pallas import tpu as pltpu
from jax.experimental.pallas import tpu_sc as plsc

B, N, S, K, C = 16, 2048, 512, 24, 128
ST = 128
NST = S // ST
NC, NS = 2, 16
NW = NC * NS
GPW = B * S // NW
HB = C // 16


def _fps_body(x_ref, y_ref, z_ref, idx_ref, cx_ref, cy_ref, cz_ref):
    X = x_ref[...]
    Y = y_ref[...]
    Z = z_ref[...]
    lane = lax.broadcasted_iota(jnp.int32, (B, N), 1)
    col = lax.broadcasted_iota(jnp.int32, (B, S), 1)

    idx_ref[...] = jnp.zeros((B, S), jnp.int32)
    cx_ref[...] = jnp.zeros((B, S), jnp.float32)
    cy_ref[...] = jnp.zeros((B, S), jnp.float32)
    cz_ref[...] = jnp.zeros((B, S), jnp.float32)

    def step(i, state):
        dists, far = state
        m = lane == far
        cx = jnp.sum(jnp.where(m, X, 0.0), axis=1, keepdims=True)
        cy = jnp.sum(jnp.where(m, Y, 0.0), axis=1, keepdims=True)
        cz = jnp.sum(jnp.where(m, Z, 0.0), axis=1, keepdims=True)
        sel = col == i
        idx_ref[...] = jnp.where(sel, far, idx_ref[...])
        cx_ref[...] = jnp.where(sel, cx, cx_ref[...])
        cy_ref[...] = jnp.where(sel, cy, cy_ref[...])
        cz_ref[...] = jnp.where(sel, cz, cz_ref[...])
        dx = X - cx
        dy = Y - cy
        dz = Z - cz
        d = dx * dx + dy * dy + dz * dz
        dists = jnp.minimum(dists, d)
        mx = jnp.max(dists, axis=1, keepdims=True)
        far = jnp.min(jnp.where(dists == mx, lane, N), axis=1, keepdims=True)
        return dists, far

    dists0 = X * 0.0 + jnp.float32(1e10)
    far0 = jnp.zeros((B, 1), dtype=jnp.int32)
    lax.fori_loop(0, S, step, (dists0, far0))


def _run_fps(xyz):
    Xc = xyz[:, :, 0]
    Yc = xyz[:, :, 1]
    Zc = xyz[:, :, 2]
    out_types = (
        jax.ShapeDtypeStruct((B, S), jnp.int32),
        jax.ShapeDtypeStruct((B, S), jnp.float32),
        jax.ShapeDtypeStruct((B, S), jnp.float32),
        jax.ShapeDtypeStruct((B, S), jnp.float32),
    )
    return pl.pallas_call(_fps_body, out_shape=out_types)(Xc, Yc, Zc)


def _knn_body(nxyz_ref, xt_ref, idx_ref, d_ref, io_ref):
    nx = nxyz_ref[0]
    xt = xt_ref[0]
    src_sq = jnp.sum(nx * nx, axis=1, keepdims=True)
    dst_sq = jnp.sum(xt * xt, axis=0, keepdims=True)
    mm = lax.dot_general(nx, xt, (((1,), (0,)), ((), ())),
                         preferred_element_type=jnp.float32)
    D = -2.0 * mm
    D = D + src_sq
    D = D + dst_sq
    d_ref[...] = D
    io_ref[...] = lax.broadcasted_iota(jnp.int32, (S, N), 1)
    colk = lax.broadcasted_iota(jnp.int32, (S, K), 1)

    idx_ref[0] = jnp.zeros((S, K), dtype=jnp.int32)

    def step(k, jprev):
        lane = io_ref[...]
        D = d_ref[...]
        D = jnp.where(lane == jprev, jnp.float32(jnp.inf), D)
        d_ref[...] = D
        mn = jnp.min(D, axis=1, keepdims=True)
        j = jnp.min(jnp.where(D == mn, lane, N), axis=1, keepdims=True)
        idx_ref[0] = jnp.where(colk == k, j, idx_ref[0])
        return j

    lax.fori_loop(0, K, step, jnp.full((S, 1), -1, jnp.int32))


def _run_knn(new_xyz, xt):
    return pl.pallas_call(
        _knn_body,
        grid=(B,),
        in_specs=[
            pl.BlockSpec((1, S, 3), lambda b: (b, 0, 0)),
            pl.BlockSpec((1, 3, N), lambda b: (b, 0, 0)),
        ],
        out_specs=pl.BlockSpec((1, S, K), lambda b: (b, 0, 0)),
        out_shape=jax.ShapeDtypeStruct((B, S, K), jnp.int32),
        scratch_shapes=[
            pltpu.VMEM((S, N), jnp.float32),
            pltpu.VMEM((S, N), jnp.int32),
        ],
    )(new_xyz, xt)


def _sc_stats(tbl, kidx3):
    mesh = plsc.VectorSubcoreMesh(core_axis_name="c", subcore_axis_name="s")

    @functools.partial(
        pl.kernel,
        mesh=mesh,
        out_type=(
            jax.ShapeDtypeStruct((NW, 16), jnp.float32),
            jax.ShapeDtypeStruct((NW, 16), jnp.float32),
        ),
        scratch_types=[
            pltpu.VMEM((GPW, K), jnp.int32),
            pltpu.VMEM((K, C), jnp.float32),
            pltpu.VMEM((K, C), jnp.float32),
            pltpu.VMEM((16,), jnp.float32),
            pltpu.VMEM((16,), jnp.float32),
            pltpu.SemaphoreType.DMA,
            pltpu.SemaphoreType.DMA,
        ],
    )
    def stats_kernel(tbl_hbm, kidx_hbm, ss_hbm, sm_hbm,
                     kidx_v, buf_a, buf_b, ss_v, sm_v, sem_a, sem_b):
        wid = lax.axis_index("s") * NC + lax.axis_index("c")
        pltpu.sync_copy(kidx_hbm.at[wid], kidx_v)
        pltpu.async_copy(tbl_hbm.at[kidx_v.at[0]], buf_a, sem_a)

        def group_stats(buf, accs):
            a_ss, a_sm = accs
            for h in range(HB):
                s = buf[0, pl.ds(h * 16, 16)]
                a_ss = a_ss + s * s
                for k in range(1, K):
                    v = buf[k, pl.ds(h * 16, 16)]
                    s = s + v
                    a_ss = a_ss + v * v
                m = s / jnp.float32(K)
                a_sm = a_sm + m * m
            return a_ss, a_sm

        def body(i, accs):
            c0 = 2 * i
            pltpu.async_copy(tbl_hbm.at[kidx_v.at[c0 + 1]], buf_b, sem_b)
            pltpu.make_async_copy(tbl_hbm.at[kidx_v.at[c0]], buf_a,
                                  sem_a).wait()
            accs = group_stats(buf_a, accs)

            @pl.when(i < GPW // 2 - 1)
            def _():
                pltpu.async_copy(tbl_hbm.at[kidx_v.at[c0 + 2]], buf_a, sem_a)

            pltpu.make_async_copy(tbl_hbm.at[kidx_v.at[c0 + 1]], buf_b,
                                  sem_b).wait()
            return group_stats(buf_b, accs)

        z = jnp.zeros((16,), jnp.float32)
        a_ss, a_sm = lax.fori_loop(0, GPW // 2, body, (z, z))
        ss_v[...] = a_ss
        sm_v[...] = a_sm
        pltpu.sync_copy(ss_v, ss_hbm.at[wid])
        pltpu.sync_copy(sm_v, sm_hbm.at[wid])

    return stats_kernel(tbl, kidx3)


def _sc_norm(tbl, kidx3, fidx3, scb, beb):
    mesh = plsc.VectorSubcoreMesh(core_axis_name="c", subcore_axis_name="s")

    @functools.partial(
        pl.kernel,
        mesh=mesh,
        out_type=jax.ShapeDtypeStruct((B * S * K, 2 * C), jnp.float32),
        scratch_types=[
            pltpu.VMEM((GPW, K), jnp.int32),
            pltpu.VMEM((2, 128), jnp.int32),
            pltpu.VMEM((GPW, C), jnp.float32),
            pltpu.VMEM((K, C), jnp.float32),
            pltpu.VMEM((K, C), jnp.float32),
            pltpu.VMEM((K, 2 * C), jnp.float32),
            pltpu.VMEM((K, 2 * C), jnp.float32),
            pltpu.VMEM((C,), jnp.float32),
            pltpu.VMEM((C,), jnp.float32),
            pltpu.SemaphoreType.DMA,
            pltpu.SemaphoreType.DMA,
            pltpu.SemaphoreType.DMA,
            pltpu.SemaphoreType.DMA,
        ],
    )
    def norm_kernel(tbl_hbm, kidx_hbm, fidx_hbm, scb_hbm, beb_hbm, out_hbm,
                    kidx_v, fidx_v, npbuf, buf_a, buf_b, ob_a, ob_b,
                    sc_v, be_v, sem_a, sem_b, sem_wa, sem_wb):
        wid = lax.axis_index("s") * NC + lax.axis_index("c")
        base = wid * GPW * K
        pltpu.sync_copy(kidx_hbm.at[wid], kidx_v)
        pltpu.sync_copy(fidx_hbm.at[wid], fidx_v)
        pltpu.sync_copy(scb_hbm.at[wid], sc_v)
        pltpu.sync_copy(beb_hbm, be_v)
        pltpu.async_copy(tbl_hbm.at[fidx_v.at[0]],
                         npbuf.at[pl.ds(0, 128)], sem_a).wait()
        pltpu.async_copy(tbl_hbm.at[fidx_v.at[1]],
                         npbuf.at[pl.ds(128, 128)], sem_a).wait()
        pltpu.async_copy(tbl_hbm.at[kidx_v.at[0]], buf_a, sem_a)

        def group_norm(c, buf, ob):
            for h in range(HB):
                hs = pl.ds(h * 16, 16)
                s = buf[0, hs]
                for k in range(1, K):
                    s = s + buf[k, hs]
                m = s / jnp.float32(K)
                sc = sc_v[hs]
                be = be_v[hs]
                for k in range(K):
                    ob[k, hs] = (buf[k, hs] - m) * sc + be
                npv = npbuf[c, hs]
                hs2 = pl.ds(C + h * 16, 16)
                for k in range(K):
                    ob[k, hs2] = npv

        def body(i, carry):
            c0 = 2 * i
            pltpu.async_copy(tbl_hbm.at[kidx_v.at[c0 + 1]], buf_b, sem_b)
            pltpu.make_async_copy(tbl_hbm.at[kidx_v.at[c0]], buf_a,
                                  sem_a).wait()

            @pl.when(i > 0)
            def _():
                pltpu.make_async_copy(
                    ob_a, out_hbm.at[pl.ds(base, K)], sem_wa).wait()

            group_norm(c0, buf_a, ob_a)
            pltpu.async_copy(ob_a, out_hbm.at[pl.ds(base + c0 * K, K)],
                             sem_wa)

            @pl.when(i < GPW // 2 - 1)
            def _():
                pltpu.async_copy(tbl_hbm.at[kidx_v.at[c0 + 2]], buf_a, sem_a)

            pltpu.make_async_copy(tbl_hbm.at[kidx_v.at[c0 + 1]], buf_b,
                                  sem_b).wait()

            @pl.when(i > 0)
            def _():
                pltpu.make_async_copy(
                    ob_b, out_hbm.at[pl.ds(base, K)], sem_wb).wait()

            group_norm(c0 + 1, buf_b, ob_b)
            pltpu.async_copy(ob_b, out_hbm.at[pl.ds(base + (c0 + 1) * K, K)],
                             sem_wb)
            return carry

        lax.fori_loop(0, GPW // 2, body, 0)
        pltpu.make_async_copy(ob_a, out_hbm.at[pl.ds(base, K)], sem_wa).wait()
        pltpu.make_async_copy(ob_b, out_hbm.at[pl.ds(base, K)], sem_wb).wait()

    return norm_kernel(tbl, kidx3, fidx3, scb, beb)


def kernel(xyz, p, affine_alpha, affine_beta):
    fps_idx, cx, cy, cz = _run_fps(xyz)
    new_xyz = jnp.stack([cx, cy, cz], axis=-1)

    xt = jnp.transpose(xyz, (0, 2, 1))
    idx = _run_knn(new_xyz, xt)

    boff = (jnp.arange(B, dtype=jnp.int32) * N)
    knn_rows = (idx + boff[:, None, None]).reshape(NW, GPW, K)
    fps_rows = (fps_idx + boff[:, None]).reshape(NW, 2, 128)

    tbl = p.reshape(B * N, C)
    ss, sm = _sc_stats(tbl, knn_rows)
    ssb = ss.reshape(B, 32).sum(axis=1)
    smb = sm.reshape(B, 32).sum(axis=1)
    var = (ssb - jnp.float32(K) * smb) / jnp.float32(S * K * C - 1)
    inv = 1.0 / (jnp.sqrt(var) + jnp.float32(1e-5))
    scb = jnp.repeat(inv, 2)[:, None] * affine_alpha.reshape(1, C)
    beb = jnp.broadcast_to(affine_beta.reshape(C), (C,))

    rows2 = _sc_norm(tbl, knn_rows, fps_rows, scb, beb)
    new_p_out = rows2.reshape(B, S, K, 2 * C)
    return (new_xyz, new_p_out)


# speedup vs baseline: 12.2020x; 1.0216x over previous
import functools

import jax
import jax.numpy as jnp
from jax import lax
from jax.experimental import pallas as pl
from jax.experimental.pallas import tpu as pltpu
from jax.experimental.pallas import tpu_sc as plsc

B, N, S, K, C = 16, 2048, 512, 24, 128
ST = 128
NST = S // ST
NC, NS = 2, 16
NW = NC * NS
GPW = B * S // NW
HB = C // 16


def _fps_body(x_ref, y_ref, z_ref, idx_ref, cx_ref, cy_ref, cz_ref):
    X = x_ref[...]
    Y = y_ref[...]
    Z = z_ref[...]
    lane = lax.broadcasted_iota(jnp.int32, (B, N), 1)
    col = lax.broadcasted_iota(jnp.int32, (B, S), 1)

    idx_ref[...] = jnp.zeros((B, S), jnp.int32)
    cx_ref[...] = jnp.zeros((B, S), jnp.float32)
    cy_ref[...] = jnp.zeros((B, S), jnp.float32)
    cz_ref[...] = jnp.zeros((B, S), jnp.float32)

    def step(i, state):
        dists, far = state
        m = lane == far
        cx = jnp.sum(jnp.where(m, X, 0.0), axis=1, keepdims=True)
        cy = jnp.sum(jnp.where(m, Y, 0.0), axis=1, keepdims=True)
        cz = jnp.sum(jnp.where(m, Z, 0.0), axis=1, keepdims=True)
        sel = col == i
        idx_ref[...] = jnp.where(sel, far, idx_ref[...])
        cx_ref[...] = jnp.where(sel, cx, cx_ref[...])
        cy_ref[...] = jnp.where(sel, cy, cy_ref[...])
        cz_ref[...] = jnp.where(sel, cz, cz_ref[...])
        dx = X - cx
        dy = Y - cy
        dz = Z - cz
        d = dx * dx + dy * dy + dz * dz
        dists = jnp.minimum(dists, d)
        mx = jnp.max(dists, axis=1, keepdims=True)
        far = jnp.min(jnp.where(dists == mx, lane, N), axis=1, keepdims=True)
        return dists, far

    dists0 = X * 0.0 + jnp.float32(1e10)
    far0 = jnp.zeros((B, 1), dtype=jnp.int32)
    lax.fori_loop(0, S, step, (dists0, far0))


def _run_fps(xyz):
    Xc = xyz[:, :, 0]
    Yc = xyz[:, :, 1]
    Zc = xyz[:, :, 2]
    out_types = (
        jax.ShapeDtypeStruct((B, S), jnp.int32),
        jax.ShapeDtypeStruct((B, S), jnp.float32),
        jax.ShapeDtypeStruct((B, S), jnp.float32),
        jax.ShapeDtypeStruct((B, S), jnp.float32),
    )
    return pl.pallas_call(_fps_body, out_shape=out_types)(Xc, Yc, Zc)


def _knn_body(nxyz_ref, xt_ref, idx_ref, d_ref, io_ref):
    nx = nxyz_ref[0]
    xt = xt_ref[0]
    src_sq = jnp.sum(nx * nx, axis=1, keepdims=True)
    dst_sq = jnp.sum(xt * xt, axis=0, keepdims=True)
    mm = lax.dot_general(nx, xt, (((1,), (0,)), ((), ())),
                         preferred_element_type=jnp.float32)
    D = -2.0 * mm
    D = D + src_sq
    D = D + dst_sq
    d_ref[...] = D
    io_ref[...] = lax.broadcasted_iota(jnp.int32, (S, N), 1)
    colk = lax.broadcasted_iota(jnp.int32, (S, K), 1)

    idx_ref[0] = jnp.zeros((S, K), dtype=jnp.int32)

    def step(k, jprev):
        lane = io_ref[...]
        D = d_ref[...]
        D = jnp.where(lane == jprev, jnp.float32(jnp.inf), D)
        d_ref[...] = D
        mn = jnp.min(D, axis=1, keepdims=True)
        j = jnp.min(jnp.where(D == mn, lane, N), axis=1, keepdims=True)
        idx_ref[0] = jnp.where(colk == k, j, idx_ref[0])
        return j

    lax.fori_loop(0, K, step, jnp.full((S, 1), -1, jnp.int32))


def _run_knn(new_xyz, xt):
    return pl.pallas_call(
        _knn_body,
        grid=(B,),
        in_specs=[
            pl.BlockSpec((1, S, 3), lambda b: (b, 0, 0)),
            pl.BlockSpec((1, 3, N), lambda b: (b, 0, 0)),
        ],
        out_specs=pl.BlockSpec((1, S, K), lambda b: (b, 0, 0)),
        out_shape=jax.ShapeDtypeStruct((B, S, K), jnp.int32),
        scratch_shapes=[
            pltpu.VMEM((S, N), jnp.float32),
            pltpu.VMEM((S, N), jnp.int32),
        ],
    )(new_xyz, xt)


def _sc_stats(tbl, kidx3):
    mesh = plsc.VectorSubcoreMesh(core_axis_name="c", subcore_axis_name="s")

    @functools.partial(
        pl.kernel,
        mesh=mesh,
        out_type=(
            jax.ShapeDtypeStruct((NW, 16), jnp.float32),
            jax.ShapeDtypeStruct((NW, 16), jnp.float32),
        ),
        scratch_types=[
            pltpu.VMEM((GPW, K), jnp.int32),
            pltpu.VMEM((K, C), jnp.float32),
            pltpu.VMEM((K, C), jnp.float32),
            pltpu.VMEM((16,), jnp.float32),
            pltpu.VMEM((16,), jnp.float32),
            pltpu.SemaphoreType.DMA,
            pltpu.SemaphoreType.DMA,
        ],
    )
    def stats_kernel(tbl_hbm, kidx_hbm, ss_hbm, sm_hbm,
                     kidx_v, buf_a, buf_b, ss_v, sm_v, sem_a, sem_b):
        wid = lax.axis_index("s") * NC + lax.axis_index("c")
        pltpu.sync_copy(kidx_hbm.at[wid], kidx_v)
        pltpu.async_copy(tbl_hbm.at[kidx_v.at[0]], buf_a, sem_a)

        def group_stats(buf, accs):
            a_ss, a_sm = accs
            for h in range(HB):
                hs = pl.ds(h * 16, 16)
                v = [buf[k, hs] for k in range(K)]
                p0 = v[0] + v[4]
                p1 = v[1] + v[5]
                p2 = v[2] + v[6]
                p3 = v[3] + v[7]
                q0 = v[8] + v[12]
                q1 = v[9] + v[13]
                q2 = v[10] + v[14]
                q3 = v[11] + v[15]
                r0 = v[16] + v[20]
                r1 = v[17] + v[21]
                r2 = v[18] + v[22]
                r3 = v[19] + v[23]
                p0 = p0 + q0
                p1 = p1 + q1
                p2 = p2 + q2
                p3 = p3 + q3
                p0 = p0 + r0
                p1 = p1 + r1
                p2 = p2 + r2
                p3 = p3 + r3
                s = (p0 + p1) + (p2 + p3)
                w0 = v[0] * v[0]
                w1 = v[1] * v[1]
                w2 = v[2] * v[2]
                w3 = v[3] * v[3]
                for k in range(4, K):
                    if k % 4 == 0:
                        w0 = w0 + v[k] * v[k]
                    elif k % 4 == 1:
                        w1 = w1 + v[k] * v[k]
                    elif k % 4 == 2:
                        w2 = w2 + v[k] * v[k]
                    else:
                        w3 = w3 + v[k] * v[k]
                m = s / jnp.float32(K)
                a_ss = a_ss + ((w0 + w1) + (w2 + w3))
                a_sm = a_sm + m * m
            return a_ss, a_sm

        def body(i, accs):
            c0 = 2 * i
            pltpu.async_copy(tbl_hbm.at[kidx_v.at[c0 + 1]], buf_b, sem_b)
            pltpu.make_async_copy(tbl_hbm.at[kidx_v.at[c0]], buf_a,
                                  sem_a).wait()
            accs = group_stats(buf_a, accs)

            @pl.when(i < GPW // 2 - 1)
            def _():
                pltpu.async_copy(tbl_hbm.at[kidx_v.at[c0 + 2]], buf_a, sem_a)

            pltpu.make_async_copy(tbl_hbm.at[kidx_v.at[c0 + 1]], buf_b,
                                  sem_b).wait()
            return group_stats(buf_b, accs)

        z = jnp.zeros((16,), jnp.float32)
        a_ss, a_sm = lax.fori_loop(0, GPW // 2, body, (z, z))
        ss_v[...] = a_ss
        sm_v[...] = a_sm
        pltpu.sync_copy(ss_v, ss_hbm.at[wid])
        pltpu.sync_copy(sm_v, sm_hbm.at[wid])

    return stats_kernel(tbl, kidx3)


def _sc_norm(tbl, kidx3, fidx3, scb, beb):
    mesh = plsc.VectorSubcoreMesh(core_axis_name="c", subcore_axis_name="s")

    @functools.partial(
        pl.kernel,
        mesh=mesh,
        out_type=jax.ShapeDtypeStruct((B * S * K, 2 * C), jnp.float32),
        scratch_types=[
            pltpu.VMEM((GPW, K), jnp.int32),
            pltpu.VMEM((2, 128), jnp.int32),
            pltpu.VMEM((GPW, C), jnp.float32),
            pltpu.VMEM((K, C), jnp.float32),
            pltpu.VMEM((K, C), jnp.float32),
            pltpu.VMEM((K, 2 * C), jnp.float32),
            pltpu.VMEM((K, 2 * C), jnp.float32),
            pltpu.VMEM((C,), jnp.float32),
            pltpu.VMEM((C,), jnp.float32),
            pltpu.SemaphoreType.DMA,
            pltpu.SemaphoreType.DMA,
            pltpu.SemaphoreType.DMA,
            pltpu.SemaphoreType.DMA,
        ],
    )
    def norm_kernel(tbl_hbm, kidx_hbm, fidx_hbm, scb_hbm, beb_hbm, out_hbm,
                    kidx_v, fidx_v, npbuf, buf_a, buf_b, ob_a, ob_b,
                    sc_v, be_v, sem_a, sem_b, sem_wa, sem_wb):
        wid = lax.axis_index("s") * NC + lax.axis_index("c")
        base = wid * GPW * K
        pltpu.sync_copy(kidx_hbm.at[wid], kidx_v)
        pltpu.sync_copy(fidx_hbm.at[wid], fidx_v)
        pltpu.sync_copy(scb_hbm.at[wid], sc_v)
        pltpu.sync_copy(beb_hbm, be_v)
        pltpu.async_copy(tbl_hbm.at[fidx_v.at[0]],
                         npbuf.at[pl.ds(0, 128)], sem_a).wait()
        pltpu.async_copy(tbl_hbm.at[fidx_v.at[1]],
                         npbuf.at[pl.ds(128, 128)], sem_a).wait()
        pltpu.async_copy(tbl_hbm.at[kidx_v.at[0]], buf_a, sem_a)

        def group_norm(c, buf, ob):
            for h in range(HB):
                hs = pl.ds(h * 16, 16)
                s = buf[0, hs]
                for k in range(1, K):
                    s = s + buf[k, hs]
                m = s / jnp.float32(K)
                sc = sc_v[hs]
                be = be_v[hs]
                for k in range(K):
                    ob[k, hs] = (buf[k, hs] - m) * sc + be
                npv = npbuf[c, hs]
                hs2 = pl.ds(C + h * 16, 16)
                for k in range(K):
                    ob[k, hs2] = npv

        def body(i, carry):
            c0 = 2 * i
            pltpu.async_copy(tbl_hbm.at[kidx_v.at[c0 + 1]], buf_b, sem_b)
            pltpu.make_async_copy(tbl_hbm.at[kidx_v.at[c0]], buf_a,
                                  sem_a).wait()

            @pl.when(i > 0)
            def _():
                pltpu.make_async_copy(
                    ob_a, out_hbm.at[pl.ds(base, K)], sem_wa).wait()

            group_norm(c0, buf_a, ob_a)
            pltpu.async_copy(ob_a, out_hbm.at[pl.ds(base + c0 * K, K)],
                             sem_wa)

            @pl.when(i < GPW // 2 - 1)
            def _():
                pltpu.async_copy(tbl_hbm.at[kidx_v.at[c0 + 2]], buf_a, sem_a)

            pltpu.make_async_copy(tbl_hbm.at[kidx_v.at[c0 + 1]], buf_b,
                                  sem_b).wait()

            @pl.when(i > 0)
            def _():
                pltpu.make_async_copy(
                    ob_b, out_hbm.at[pl.ds(base, K)], sem_wb).wait()

            group_norm(c0 + 1, buf_b, ob_b)
            pltpu.async_copy(ob_b, out_hbm.at[pl.ds(base + (c0 + 1) * K, K)],
                             sem_wb)
            return carry

        lax.fori_loop(0, GPW // 2, body, 0)
        pltpu.make_async_copy(ob_a, out_hbm.at[pl.ds(base, K)], sem_wa).wait()
        pltpu.make_async_copy(ob_b, out_hbm.at[pl.ds(base, K)], sem_wb).wait()

    return norm_kernel(tbl, kidx3, fidx3, scb, beb)


def kernel(xyz, p, affine_alpha, affine_beta):
    fps_idx, cx, cy, cz = _run_fps(xyz)
    new_xyz = jnp.stack([cx, cy, cz], axis=-1)

    xt = jnp.transpose(xyz, (0, 2, 1))
    idx = _run_knn(new_xyz, xt)

    boff = (jnp.arange(B, dtype=jnp.int32) * N)
    knn_rows = (idx + boff[:, None, None]).reshape(NW, GPW, K)
    fps_rows = (fps_idx + boff[:, None]).reshape(NW, 2, 128)

    tbl = p.reshape(B * N, C)
    ss, sm = _sc_stats(tbl, knn_rows)
    ssb = ss.reshape(B, 32).sum(axis=1)
    smb = sm.reshape(B, 32).sum(axis=1)
    var = (ssb - jnp.float32(K) * smb) / jnp.float32(S * K * C - 1)
    inv = 1.0 / (jnp.sqrt(var) + jnp.float32(1e-5))
    scb = jnp.repeat(inv, 2)[:, None] * affine_alpha.reshape(1, C)
    beb = jnp.broadcast_to(affine_beta.reshape(C), (C,))

    rows2 = _sc_norm(tbl, knn_rows, fps_rows, scb, beb)
    new_p_out = rows2.reshape(B, S, K, 2 * C)
    return (new_xyz, new_p_out)


# speedup vs baseline: 12.2506x; 1.0040x over previous
import functools

import jax
import jax.numpy as jnp
from jax import lax
from jax.experimental import pallas as pl
from jax.experimental.pallas import tpu as pltpu
from jax.experimental.pallas import tpu_sc as plsc

B, N, S, K, C = 16, 2048, 512, 24, 128
ST = 128
NST = S // ST
NC, NS = 2, 16
NW = NC * NS
GPW = B * S // NW
HB = C // 16


def _fps_body(x_ref, y_ref, z_ref, idx_ref, cx_ref, cy_ref, cz_ref):
    X = x_ref[...]
    Y = y_ref[...]
    Z = z_ref[...]
    lane = lax.broadcasted_iota(jnp.int32, (B, N), 1)
    col = lax.broadcasted_iota(jnp.int32, (B, S), 1)

    idx_ref[...] = jnp.zeros((B, S), jnp.int32)
    cx_ref[...] = jnp.zeros((B, S), jnp.float32)
    cy_ref[...] = jnp.zeros((B, S), jnp.float32)
    cz_ref[...] = jnp.zeros((B, S), jnp.float32)

    def step(i, state):
        dists, far = state
        m = lane == far
        cx = jnp.sum(jnp.where(m, X, 0.0), axis=1, keepdims=True)
        cy = jnp.sum(jnp.where(m, Y, 0.0), axis=1, keepdims=True)
        cz = jnp.sum(jnp.where(m, Z, 0.0), axis=1, keepdims=True)
        sel = col == i
        idx_ref[...] = jnp.where(sel, far, idx_ref[...])
        cx_ref[...] = jnp.where(sel, cx, cx_ref[...])
        cy_ref[...] = jnp.where(sel, cy, cy_ref[...])
        cz_ref[...] = jnp.where(sel, cz, cz_ref[...])
        dx = X - cx
        dy = Y - cy
        dz = Z - cz
        d = dx * dx + dy * dy + dz * dz
        dists = jnp.minimum(dists, d)
        mx = jnp.max(dists, axis=1, keepdims=True)
        far = jnp.min(jnp.where(dists == mx, lane, N), axis=1, keepdims=True)
        return dists, far

    dists0 = X * 0.0 + jnp.float32(1e10)
    far0 = jnp.zeros((B, 1), dtype=jnp.int32)
    lax.fori_loop(0, S, step, (dists0, far0))


def _run_fps(xyz):
    Xc = xyz[:, :, 0]
    Yc = xyz[:, :, 1]
    Zc = xyz[:, :, 2]
    out_types = (
        jax.ShapeDtypeStruct((B, S), jnp.int32),
        jax.ShapeDtypeStruct((B, S), jnp.float32),
        jax.ShapeDtypeStruct((B, S), jnp.float32),
        jax.ShapeDtypeStruct((B, S), jnp.float32),
    )
    return pl.pallas_call(_fps_body, out_shape=out_types)(Xc, Yc, Zc)


def _knn_body(nxyz_ref, xt_ref, idx_ref, d_ref, io_ref):
    nx = nxyz_ref[0]
    xt = xt_ref[0]
    src_sq = jnp.sum(nx * nx, axis=1, keepdims=True)
    dst_sq = jnp.sum(xt * xt, axis=0, keepdims=True)
    mm = lax.dot_general(nx, xt, (((1,), (0,)), ((), ())),
                         preferred_element_type=jnp.float32)
    D = -2.0 * mm
    D = D + src_sq
    D = D + dst_sq
    d_ref[...] = D
    io_ref[...] = lax.broadcasted_iota(jnp.int32, (S, N), 1)
    colk = lax.broadcasted_iota(jnp.int32, (S, K), 1)

    idx_ref[0] = jnp.zeros((S, K), dtype=jnp.int32)

    def step(k, jprev):
        lane = io_ref[...]
        D = d_ref[...]
        D = jnp.where(lane == jprev, jnp.float32(jnp.inf), D)
        d_ref[...] = D
        mn = jnp.min(D, axis=1, keepdims=True)
        j = jnp.min(jnp.where(D == mn, lane, N), axis=1, keepdims=True)
        idx_ref[0] = jnp.where(colk == k, j, idx_ref[0])
        return j

    lax.fori_loop(0, K, step, jnp.full((S, 1), -1, jnp.int32))


def _run_knn(new_xyz, xt):
    return pl.pallas_call(
        _knn_body,
        grid=(B,),
        in_specs=[
            pl.BlockSpec((1, S, 3), lambda b: (b, 0, 0)),
            pl.BlockSpec((1, 3, N), lambda b: (b, 0, 0)),
        ],
        out_specs=pl.BlockSpec((1, S, K), lambda b: (b, 0, 0)),
        out_shape=jax.ShapeDtypeStruct((B, S, K), jnp.int32),
        scratch_shapes=[
            pltpu.VMEM((S, N), jnp.float32),
            pltpu.VMEM((S, N), jnp.int32),
        ],
    )(new_xyz, xt)


def _sc_stats(tbl, kidx3):
    mesh = plsc.VectorSubcoreMesh(core_axis_name="c", subcore_axis_name="s")

    @functools.partial(
        pl.kernel,
        mesh=mesh,
        out_type=(
            jax.ShapeDtypeStruct((NW, 16), jnp.float32),
            jax.ShapeDtypeStruct((NW, 16), jnp.float32),
        ),
        scratch_types=[
            pltpu.VMEM((GPW, K), jnp.int32),
            pltpu.VMEM((K, C), jnp.float32),
            pltpu.VMEM((K, C), jnp.float32),
            pltpu.VMEM((16,), jnp.float32),
            pltpu.VMEM((16,), jnp.float32),
            pltpu.SemaphoreType.DMA,
            pltpu.SemaphoreType.DMA,
        ],
    )
    def stats_kernel(tbl_hbm, kidx_hbm, ss_hbm, sm_hbm,
                     kidx_v, buf_a, buf_b, ss_v, sm_v, sem_a, sem_b):
        wid = lax.axis_index("s") * NC + lax.axis_index("c")
        pltpu.sync_copy(kidx_hbm.at[wid], kidx_v)
        pltpu.async_copy(tbl_hbm.at[kidx_v.at[0]], buf_a, sem_a)

        def group_stats(buf, accs):
            a_ss, a_sm = accs
            for h in range(HB):
                hs = pl.ds(h * 16, 16)
                v = [buf[k, hs] for k in range(K)]
                p0 = v[0] + v[4]
                p1 = v[1] + v[5]
                p2 = v[2] + v[6]
                p3 = v[3] + v[7]
                q0 = v[8] + v[12]
                q1 = v[9] + v[13]
                q2 = v[10] + v[14]
                q3 = v[11] + v[15]
                r0 = v[16] + v[20]
                r1 = v[17] + v[21]
                r2 = v[18] + v[22]
                r3 = v[19] + v[23]
                p0 = p0 + q0
                p1 = p1 + q1
                p2 = p2 + q2
                p3 = p3 + q3
                p0 = p0 + r0
                p1 = p1 + r1
                p2 = p2 + r2
                p3 = p3 + r3
                s = (p0 + p1) + (p2 + p3)
                w0 = v[0] * v[0]
                w1 = v[1] * v[1]
                w2 = v[2] * v[2]
                w3 = v[3] * v[3]
                for k in range(4, K):
                    if k % 4 == 0:
                        w0 = w0 + v[k] * v[k]
                    elif k % 4 == 1:
                        w1 = w1 + v[k] * v[k]
                    elif k % 4 == 2:
                        w2 = w2 + v[k] * v[k]
                    else:
                        w3 = w3 + v[k] * v[k]
                m = s / jnp.float32(K)
                a_ss = a_ss + ((w0 + w1) + (w2 + w3))
                a_sm = a_sm + m * m
            return a_ss, a_sm

        def body(i, accs):
            c0 = 2 * i
            pltpu.async_copy(tbl_hbm.at[kidx_v.at[c0 + 1]], buf_b, sem_b)
            pltpu.make_async_copy(tbl_hbm.at[kidx_v.at[c0]], buf_a,
                                  sem_a).wait()
            accs = group_stats(buf_a, accs)

            @pl.when(i < GPW // 2 - 1)
            def _():
                pltpu.async_copy(tbl_hbm.at[kidx_v.at[c0 + 2]], buf_a, sem_a)

            pltpu.make_async_copy(tbl_hbm.at[kidx_v.at[c0 + 1]], buf_b,
                                  sem_b).wait()
            return group_stats(buf_b, accs)

        z = jnp.zeros((16,), jnp.float32)
        a_ss, a_sm = lax.fori_loop(0, GPW // 2, body, (z, z))
        ss_v[...] = a_ss
        sm_v[...] = a_sm
        pltpu.sync_copy(ss_v, ss_hbm.at[wid])
        pltpu.sync_copy(sm_v, sm_hbm.at[wid])

    return stats_kernel(tbl, kidx3)


def _sc_norm(tbl, kidx3, fidx3, scb, beb):
    mesh = plsc.VectorSubcoreMesh(core_axis_name="c", subcore_axis_name="s")

    @functools.partial(
        pl.kernel,
        mesh=mesh,
        out_type=jax.ShapeDtypeStruct((B * S * K, 2 * C), jnp.float32),
        scratch_types=[
            pltpu.VMEM((GPW, K), jnp.int32),
            pltpu.VMEM((2, 128), jnp.int32),
            pltpu.VMEM((GPW, C), jnp.float32),
            pltpu.VMEM((K, C), jnp.float32),
            pltpu.VMEM((K, C), jnp.float32),
            pltpu.VMEM((K, 2 * C), jnp.float32),
            pltpu.VMEM((K, 2 * C), jnp.float32),
            pltpu.VMEM((C,), jnp.float32),
            pltpu.VMEM((C,), jnp.float32),
            pltpu.SemaphoreType.DMA,
            pltpu.SemaphoreType.DMA,
            pltpu.SemaphoreType.DMA,
            pltpu.SemaphoreType.DMA,
        ],
    )
    def norm_kernel(tbl_hbm, kidx_hbm, fidx_hbm, scb_hbm, beb_hbm, out_hbm,
                    kidx_v, fidx_v, npbuf, buf_a, buf_b, ob_a, ob_b,
                    sc_v, be_v, sem_a, sem_b, sem_wa, sem_wb):
        wid = lax.axis_index("s") * NC + lax.axis_index("c")
        base = wid * GPW * K
        pltpu.sync_copy(kidx_hbm.at[wid], kidx_v)
        pltpu.sync_copy(fidx_hbm.at[wid], fidx_v)
        pltpu.sync_copy(scb_hbm.at[wid], sc_v)
        pltpu.sync_copy(beb_hbm, be_v)
        pltpu.async_copy(tbl_hbm.at[fidx_v.at[0]],
                         npbuf.at[pl.ds(0, 128)], sem_a).wait()
        pltpu.async_copy(tbl_hbm.at[fidx_v.at[1]],
                         npbuf.at[pl.ds(128, 128)], sem_a).wait()
        pltpu.async_copy(tbl_hbm.at[kidx_v.at[0]], buf_a, sem_a)

        def group_norm(c, buf, ob):
            for h in range(HB):
                hs = pl.ds(h * 16, 16)
                v = [buf[k, hs] for k in range(K)]
                p0 = (v[0] + v[4]) + (v[8] + v[12])
                p1 = (v[1] + v[5]) + (v[9] + v[13])
                p2 = (v[2] + v[6]) + (v[10] + v[14])
                p3 = (v[3] + v[7]) + (v[11] + v[15])
                p0 = p0 + (v[16] + v[20])
                p1 = p1 + (v[17] + v[21])
                p2 = p2 + (v[18] + v[22])
                p3 = p3 + (v[19] + v[23])
                s = (p0 + p1) + (p2 + p3)
                m = s / jnp.float32(K)
                sc = sc_v[hs]
                be = be_v[hs]
                for k in range(K):
                    ob[k, hs] = (v[k] - m) * sc + be
                npv = npbuf[c, hs]
                hs2 = pl.ds(C + h * 16, 16)
                for k in range(K):
                    ob[k, hs2] = npv

        def body(i, carry):
            c0 = 2 * i
            pltpu.async_copy(tbl_hbm.at[kidx_v.at[c0 + 1]], buf_b, sem_b)
            pltpu.make_async_copy(tbl_hbm.at[kidx_v.at[c0]], buf_a,
                                  sem_a).wait()

            @pl.when(i > 0)
            def _():
                pltpu.make_async_copy(
                    ob_a, out_hbm.at[pl.ds(base, K)], sem_wa).wait()

            group_norm(c0, buf_a, ob_a)
            pltpu.async_copy(ob_a, out_hbm.at[pl.ds(base + c0 * K, K)],
                             sem_wa)

            @pl.when(i < GPW // 2 - 1)
            def _():
                pltpu.async_copy(tbl_hbm.at[kidx_v.at[c0 + 2]], buf_a, sem_a)

            pltpu.make_async_copy(tbl_hbm.at[kidx_v.at[c0 + 1]], buf_b,
                                  sem_b).wait()

            @pl.when(i > 0)
            def _():
                pltpu.make_async_copy(
                    ob_b, out_hbm.at[pl.ds(base, K)], sem_wb).wait()

            group_norm(c0 + 1, buf_b, ob_b)
            pltpu.async_copy(ob_b, out_hbm.at[pl.ds(base + (c0 + 1) * K, K)],
                             sem_wb)
            return carry

        lax.fori_loop(0, GPW // 2, body, 0)
        pltpu.make_async_copy(ob_a, out_hbm.at[pl.ds(base, K)], sem_wa).wait()
        pltpu.make_async_copy(ob_b, out_hbm.at[pl.ds(base, K)], sem_wb).wait()

    return norm_kernel(tbl, kidx3, fidx3, scb, beb)


def kernel(xyz, p, affine_alpha, affine_beta):
    fps_idx, cx, cy, cz = _run_fps(xyz)
    new_xyz = jnp.stack([cx, cy, cz], axis=-1)

    xt = jnp.transpose(xyz, (0, 2, 1))
    idx = _run_knn(new_xyz, xt)

    boff = (jnp.arange(B, dtype=jnp.int32) * N)
    knn_rows = (idx + boff[:, None, None]).reshape(NW, GPW, K)
    fps_rows = (fps_idx + boff[:, None]).reshape(NW, 2, 128)

    tbl = p.reshape(B * N, C)
    ss, sm = _sc_stats(tbl, knn_rows)
    ssb = ss.reshape(B, 32).sum(axis=1)
    smb = sm.reshape(B, 32).sum(axis=1)
    var = (ssb - jnp.float32(K) * smb) / jnp.float32(S * K * C - 1)
    inv = 1.0 / (jnp.sqrt(var) + jnp.float32(1e-5))
    scb = jnp.repeat(inv, 2)[:, None] * affine_alpha.reshape(1, C)
    beb = jnp.broadcast_to(affine_beta.reshape(C), (C,))

    rows2 = _sc_norm(tbl, knn_rows, fps_rows, scb, beb)
    new_p_out = rows2.reshape(B, S, K, 2 * C)
    return (new_xyz, new_p_out)
